# Initial kernel scaffold; baseline (speedup 1.0000x reference)
import functools

import jax
import jax.numpy as jnp
from jax import lax
from jax.experimental import pallas as pl
from jax.experimental.pallas import tpu as pltpu
from jax.experimental.pallas import tpu_sc as plsc

B = 16384
H = 50
D = 16
NC = 2
NS = 16
NW = NC * NS
BAGS_PER_W = B // NW
BAGS_PER_ROW = 2
IDS_PER_ROW = BAGS_PER_ROW * H
ROWS_PER_W = BAGS_PER_W // BAGS_PER_ROW

_mesh = plsc.VectorSubcoreMesh(core_axis_name="c", subcore_axis_name="s")


def _tree_sum(vals):
    while len(vals) > 1:
        nxt = [a + b for a, b in zip(vals[::2], vals[1::2])]
        if len(vals) % 2:
            nxt.append(vals[-1])
        vals = nxt
    return vals[0]


@functools.partial(
    pl.kernel,
    out_type=jax.ShapeDtypeStruct((B, D), jnp.float32),
    mesh=_mesh,
    scratch_types=[
        pltpu.VMEM((ROWS_PER_W, IDS_PER_ROW), jnp.int32),
        pltpu.VMEM((IDS_PER_ROW, D), jnp.float32),
        pltpu.VMEM((BAGS_PER_W, D), jnp.float32),
        pltpu.SemaphoreType.DMA,
    ],
)
def _emb_bag(ids_hbm, table_hbm, out_hbm, idx_v, rows_v, out_v, sem):
    wid = lax.axis_index("s") * NC + lax.axis_index("c")
    pltpu.sync_copy(ids_hbm.at[pl.ds(wid * ROWS_PER_W, ROWS_PER_W)], idx_v)

    def body(j, carry):
        pltpu.async_copy(table_hbm.at[idx_v.at[j]], rows_v, sem).wait()
        for b in range(BAGS_PER_ROW):
            acc = _tree_sum([rows_v[b * H + h] for h in range(H)])
            out_v[BAGS_PER_ROW * j + b] = acc
        return carry

    lax.fori_loop(0, ROWS_PER_W, body, 0)
    pltpu.sync_copy(out_v, out_hbm.at[pl.ds(wid * BAGS_PER_W, BAGS_PER_W)])


def kernel(ids, table):
    ids2 = ids.reshape(B * H // IDS_PER_ROW, IDS_PER_ROW)
    return _emb_bag(ids2, table)


# speedup vs baseline: 1.2799x; 1.2799x over previous
import functools

import jax
import jax.numpy as jnp
from jax import lax
from jax.experimental import pallas as pl
from jax.experimental.pallas import tpu as pltpu
from jax.experimental.pallas import tpu_sc as plsc

B = 16384
H = 50
D = 16
NC = 2
NS = 16
NW = NC * NS
BAGS_PER_W = B // NW
BAGS_PER_ROW = 2
IDS_PER_ROW = BAGS_PER_ROW * H
ROWS_PER_W = BAGS_PER_W // BAGS_PER_ROW

_mesh = plsc.VectorSubcoreMesh(core_axis_name="c", subcore_axis_name="s")


def _tree_sum(vals):
    while len(vals) > 1:
        nxt = [a + b for a, b in zip(vals[::2], vals[1::2])]
        if len(vals) % 2:
            nxt.append(vals[-1])
        vals = nxt
    return vals[0]


@functools.partial(
    pl.kernel,
    out_type=jax.ShapeDtypeStruct((B, D), jnp.float32),
    mesh=_mesh,
    scratch_types=[
        pltpu.VMEM((ROWS_PER_W, IDS_PER_ROW), jnp.int32),
        pltpu.VMEM((IDS_PER_ROW, D), jnp.float32),
        pltpu.VMEM((BAGS_PER_W, D), jnp.float32),
        pltpu.SemaphoreType.DMA,
    ],
    compiler_params=pltpu.CompilerParams(use_tc_tiling_on_sc=False),
)
def _emb_bag(ids_hbm, table_hbm, out_hbm, idx_v, rows_v, out_v, sem):
    wid = lax.axis_index("s") * NC + lax.axis_index("c")
    pltpu.sync_copy(ids_hbm.at[pl.ds(wid * ROWS_PER_W, ROWS_PER_W)], idx_v)

    def body(j, carry):
        pltpu.async_copy(table_hbm.at[idx_v.at[j]], rows_v, sem).wait()
        for b in range(BAGS_PER_ROW):
            acc = _tree_sum([rows_v[b * H + h] for h in range(H)])
            out_v[BAGS_PER_ROW * j + b] = acc
        return carry

    lax.fori_loop(0, ROWS_PER_W, body, 0)
    pltpu.sync_copy(out_v, out_hbm.at[pl.ds(wid * BAGS_PER_W, BAGS_PER_W)])


def kernel(ids, table):
    ids2 = ids.reshape(B * H // IDS_PER_ROW, IDS_PER_ROW)
    return _emb_bag(ids2, table)


# speedup vs baseline: 1.6453x; 1.2854x over previous
import functools

import jax
import jax.numpy as jnp
from jax import lax
from jax.experimental import pallas as pl
from jax.experimental.pallas import tpu as pltpu
from jax.experimental.pallas import tpu_sc as plsc

B = 16384
H = 50
D = 16
NC = 2
NS = 16
NW = NC * NS
BAGS_PER_W = B // NW
BAGS_PER_ROW = 2
IDS_PER_ROW = BAGS_PER_ROW * H
ROWS_PER_W = BAGS_PER_W // BAGS_PER_ROW

_mesh = plsc.VectorSubcoreMesh(core_axis_name="c", subcore_axis_name="s")


def _tree_sum(vals):
    while len(vals) > 1:
        nxt = [a + b for a, b in zip(vals[::2], vals[1::2])]
        if len(vals) % 2:
            nxt.append(vals[-1])
        vals = nxt
    return vals[0]


NBUF = 4


@functools.partial(
    pl.kernel,
    out_type=jax.ShapeDtypeStruct((B, D), jnp.float32),
    mesh=_mesh,
    scratch_types=[
        pltpu.VMEM((ROWS_PER_W, IDS_PER_ROW), jnp.int32),
        pltpu.VMEM((NBUF, IDS_PER_ROW, D), jnp.float32),
        pltpu.VMEM((BAGS_PER_W, D), jnp.float32),
        [pltpu.SemaphoreType.DMA] * NBUF,
    ],
    compiler_params=pltpu.CompilerParams(use_tc_tiling_on_sc=False),
)
def _emb_bag(ids_hbm, table_hbm, out_hbm, idx_v, rows_v, out_v, sems):
    wid = lax.axis_index("s") * NC + lax.axis_index("c")
    pltpu.sync_copy(ids_hbm.at[pl.ds(wid * ROWS_PER_W, ROWS_PER_W)], idx_v)

    for b in range(NBUF):
        pltpu.async_copy(table_hbm.at[idx_v.at[b]], rows_v.at[b], sems[b])

    def chunk(i, carry):
        j0 = i * NBUF
        for b in range(NBUF):
            j = j0 + b
            pltpu.make_async_copy(
                table_hbm.at[idx_v.at[j]], rows_v.at[b], sems[b]
            ).wait()
            for g in range(BAGS_PER_ROW):
                acc = _tree_sum([rows_v[b, g * H + h] for h in range(H)])
                out_v[BAGS_PER_ROW * j + g] = acc
            jn = j + NBUF

            @pl.when(jn < ROWS_PER_W)
            def _():
                pltpu.async_copy(table_hbm.at[idx_v.at[jn]], rows_v.at[b], sems[b])

        return carry

    lax.fori_loop(0, ROWS_PER_W // NBUF, chunk, 0)
    pltpu.sync_copy(out_v, out_hbm.at[pl.ds(wid * BAGS_PER_W, BAGS_PER_W)])


def kernel(ids, table):
    ids2 = ids.reshape(B * H // IDS_PER_ROW, IDS_PER_ROW)
    return _emb_bag(ids2, table)


# speedup vs baseline: 1.7119x; 1.0405x over previous
import functools

import jax
import jax.numpy as jnp
from jax import lax
from jax.experimental import pallas as pl
from jax.experimental.pallas import tpu as pltpu
from jax.experimental.pallas import tpu_sc as plsc

B = 16384
H = 50
D = 16
NC = 2
NS = 16
NW = NC * NS
BAGS_PER_W = B // NW
IDS_PER_STREAM = 800
BAGS_PER_STREAM = IDS_PER_STREAM // H
NSTREAM = BAGS_PER_W // BAGS_PER_STREAM
NBUF = 4

_mesh = plsc.VectorSubcoreMesh(core_axis_name="c", subcore_axis_name="s")


def _tree_sum(vals):
    while len(vals) > 1:
        nxt = [a + b for a, b in zip(vals[::2], vals[1::2])]
        if len(vals) % 2:
            nxt.append(vals[-1])
        vals = nxt
    return vals[0]


@functools.partial(
    pl.kernel,
    out_type=jax.ShapeDtypeStruct((B, D), jnp.float32),
    mesh=_mesh,
    scratch_types=[
        pltpu.VMEM((NSTREAM, IDS_PER_STREAM), jnp.int32),
        pltpu.VMEM((NBUF, IDS_PER_STREAM, D), jnp.float32),
        pltpu.VMEM((BAGS_PER_W, D), jnp.float32),
        [pltpu.SemaphoreType.DMA] * NBUF,
    ],
    compiler_params=pltpu.CompilerParams(use_tc_tiling_on_sc=False),
)
def _emb_bag(ids_hbm, table_hbm, out_hbm, idx_v, rows_v, out_v, sems):
    wid = lax.axis_index("s") * NC + lax.axis_index("c")
    pltpu.sync_copy(ids_hbm.at[pl.ds(wid * NSTREAM, NSTREAM)], idx_v)

    for b in range(NBUF):
        pltpu.async_copy(table_hbm.at[idx_v.at[b]], rows_v.at[b], sems[b])

    def step(s, b):
        pltpu.make_async_copy(
            table_hbm.at[idx_v.at[s]], rows_v.at[b], sems[b]
        ).wait()

        def red(k, carry):
            acc = _tree_sum([rows_v[b, k * H + h] for h in range(H)])
            out_v[BAGS_PER_STREAM * s + k] = acc
            return carry

        lax.fori_loop(0, BAGS_PER_STREAM, red, 0)

    def chunk(i, carry):
        s0 = i * NBUF
        for b in range(NBUF):
            s = s0 + b
            step(s, b)
            pltpu.async_copy(
                table_hbm.at[idx_v.at[s + NBUF]], rows_v.at[b], sems[b]
            )
        return carry

    lax.fori_loop(0, NSTREAM // NBUF - 1, chunk, 0)
    for b in range(NBUF):
        step(NSTREAM - NBUF + b, b)

    pltpu.sync_copy(out_v, out_hbm.at[pl.ds(wid * BAGS_PER_W, BAGS_PER_W)])


def kernel(ids, table):
    ids2 = ids.reshape(B * H // IDS_PER_STREAM, IDS_PER_STREAM)
    return _emb_bag(ids2, table)
